# baseline (speedup 1.0000x reference)
import functools

import jax
import jax.numpy as jnp
from jax import lax
from jax.experimental import pallas as pl
from jax.experimental.pallas import tpu as pltpu
from jax.experimental.pallas import tpu_sc as plsc

B = 16384
D = 64
L = 16
NC = 2
NS = 16
NW = NC * NS
BW = B // NW
GCH = 128
NG = BW // GCH


def _cf_body(uidx, iidx, utab, itab, out,
             uidx_v, iidx_v, urows, irows, out_v, sem):
    wid = lax.axis_index("s") * NC + lax.axis_index("c")
    base = wid * BW

    pltpu.sync_copy(uidx.at[pl.ds(base, BW)], uidx_v)
    pltpu.sync_copy(iidx.at[pl.ds(base, BW)], iidx_v)

    copies = []
    for g in range(NG):
        sl = pl.ds(g * GCH, GCH)
        copies.append(pltpu.async_copy(utab.at[uidx_v.at[sl]], urows.at[sl], sem))
        copies.append(pltpu.async_copy(itab.at[iidx_v.at[sl]], irows.at[sl], sem))
    for c in copies:
        c.wait()

    iota = lax.iota(jnp.int32, L)
    zero = jnp.zeros((L,), jnp.float32)

    def chunk_body(c, carry):
        res = zero
        b0 = c * L
        for j in range(L):
            b = b0 + j
            acc = urows[b, pl.ds(0, L)] * irows[b, pl.ds(0, L)]
            for k in range(1, D // L):
                acc = acc + urows[b, pl.ds(k * L, L)] * irows[b, pl.ds(k * L, L)]
            res = jnp.where(iota == j, jnp.sum(acc), res)
        out_v[pl.ds(b0, L)] = res
        return carry

    lax.fori_loop(0, BW // L, chunk_body, 0)

    pltpu.sync_copy(out_v, out.at[pl.ds(base, BW)])


_cf_kernel = functools.partial(
    pl.kernel,
    out_type=jax.ShapeDtypeStruct((B,), jnp.float32),
    mesh=plsc.VectorSubcoreMesh(core_axis_name="c", subcore_axis_name="s"),
    compiler_params=pltpu.CompilerParams(
        needs_layout_passes=False, use_tc_tiling_on_sc=False),
    scratch_types=[
        pltpu.VMEM((BW,), jnp.int32),
        pltpu.VMEM((BW,), jnp.int32),
        pltpu.VMEM((BW, D), jnp.float32),
        pltpu.VMEM((BW, D), jnp.float32),
        pltpu.VMEM((BW,), jnp.float32),
        pltpu.SemaphoreType.DMA,
    ],
)(_cf_body)


@jax.jit
def kernel(user_input, item_input, user_table, item_table):
    return _cf_kernel(user_input.astype(jnp.int32),
                      item_input.astype(jnp.int32),
                      user_table, item_table)
